# baseline (speedup 1.0000x reference)
import functools

import jax
import jax.numpy as jnp
from jax import lax
from jax.experimental import pallas as pl
from jax.experimental.pallas import tpu as pltpu
from jax.experimental.pallas import tpu_sc as plsc

T_ALL, N = 20, 4096
NSTEP = 19
INPUT_DIM, HIDDEN, MEDIATE, OUT_DIM, SOCIAL = 2, 128, 128, 2, 16
N_SIZE = 2
CELL = 0.3
GRID = 256
PAD = 8

NC, NS = 2, 16
NW = NC * NS
ROWS_PER_TILE = 257
R_TAB = NS * ROWS_PER_TILE
DUMMY = N
SC_SCAT = N // NS
SC_GATH = N // NW


LUT_ROWS = 513
G_EMPTY = GRID * GRID


def _cell_keys(X, part_masks):
    Xs = X[:NSTEP]
    m = part_masks[:NSTEP]
    margin = 2 * N_SIZE * CELL
    x_min = Xs[:, :, 0].min(axis=1, keepdims=True) - margin
    y_min = Xs[:, :, 1].min(axis=1, keepdims=True) - margin
    px = jnp.floor((Xs[:, :, 0] - x_min) / CELL).astype(jnp.int32) * m.astype(jnp.int32)
    py = jnp.floor((Xs[:, :, 1] - y_min) / CELL).astype(jnp.int32) * m.astype(jnp.int32)
    px = jnp.clip(px, 0, GRID - 1)
    py = jnp.clip(py, 0, GRID - 1)
    key = px * GRID + py
    g = key - GRID - 1
    geff = jnp.where((m != 0) & (g >= 0), g, jnp.int32(G_EMPTY))
    return key, geff


def _route_body(key_hbm, geff_hbm, zlut_hbm, cid_hbm, gid_hbm,
                lut, keys, gbuf, cidb, gidb):
    c = lax.axis_index("c")
    s = lax.axis_index("s")
    wid = c * NS + s

    @pl.when(wid < NSTEP)
    def _():
        pltpu.sync_copy(zlut_hbm, lut)
        pltpu.sync_copy(key_hbm.at[wid], keys)
        pltpu.sync_copy(geff_hbm.at[wid], gbuf)

        def _split(idx):
            return [lax.shift_right_logical(idx, 7), lax.bitwise_and(idx, 127)]

        def scat(j, carry):
            idx = keys[j >> 3, pl.ds((j & 7) * 16, 16)]
            ids = lax.iota(jnp.int32, 16) + (j * 16 + 1)
            plsc.store_scatter(lut, _split(idx), ids)
            return carry

        lax.fori_loop(0, N // 16, scat, 0)

        def gath(j, carry):
            r, cs = j >> 3, (j & 7) * 16
            idx = keys[r, pl.ds(cs, 16)]
            cidb[r, pl.ds(cs, 16)] = plsc.load_gather(lut, _split(idx)) - 1
            gidx = gbuf[r, pl.ds(cs, 16)]
            rep = plsc.load_gather(lut, _split(gidx))
            gidb[r, pl.ds(cs, 16)] = jnp.where(rep == 0, jnp.int32(DUMMY), rep - 1)
            return carry

        lax.fori_loop(0, N // 16, gath, 0)
        pltpu.sync_copy(cidb, cid_hbm.at[wid])
        pltpu.sync_copy(gidb, gid_hbm.at[wid])


@functools.cache
def _get_route():
    return pl.kernel(
        _route_body,
        out_type=[jax.ShapeDtypeStruct((NSTEP, N // 128, 128), jnp.int32),
                  jax.ShapeDtypeStruct((NSTEP, N // 128, 128), jnp.int32)],
        mesh=plsc.VectorSubcoreMesh(core_axis_name="c", subcore_axis_name="s",
                                    num_cores=NC, num_subcores=NS),
        scratch_types=[
            pltpu.VMEM((LUT_ROWS, 128), jnp.int32),
            pltpu.VMEM((N // 128, 128), jnp.int32),
            pltpu.VMEM((N // 128, 128), jnp.int32),
            pltpu.VMEM((N // 128, 128), jnp.int32),
            pltpu.VMEM((N // 128, 128), jnp.int32),
        ],
        compiler_params=pltpu.CompilerParams(needs_layout_passes=False),
        name="social_route_sc",
    )


def _precompute_indices(X, part_masks):
    key, geff = _cell_keys(X, part_masks)
    zlut = jnp.zeros((LUT_ROWS, 128), jnp.int32)
    cid, gid = _get_route()(key.reshape(NSTEP, N // 128, 128),
                            geff.reshape(NSTEP, N // 128, 128), zlut)
    return cid, gid


HHALF = HIDDEN // NC


def _pool_body(ha_hbm, hb_hbm, cid_hbm, gid_hbm, va_hbm, vb_hbm,
               h_buf, vals_buf, cidx, gidx, table, sem):
    c = lax.axis_index("c")
    s = lax.axis_index("s")
    def zf(j, carry):
        vals_buf[j >> 2, pl.ds((j & 3) * 16, 16)] = jnp.zeros((16,), jnp.float32)
        return carry

    lax.fori_loop(0, SC_SCAT * HHALF // 16, zf, 0)
    pltpu.sync_copy(vals_buf, table.at[pl.ds(s * ROWS_PER_TILE, SC_SCAT)])
    pltpu.sync_copy(vals_buf.at[pl.ds(0, 1)],
                    table.at[pl.ds(s * ROWS_PER_TILE + SC_SCAT, 1)])
    @pl.when(c == 0)
    def _():
        pltpu.sync_copy(ha_hbm.at[pl.ds(s * SC_SCAT, SC_SCAT)], h_buf)

    @pl.when(c == 1)
    def _():
        pltpu.sync_copy(hb_hbm.at[pl.ds(s * SC_SCAT, SC_SCAT)], h_buf)

    pltpu.sync_copy(cid_hbm.at[pl.ds(s * 2, 2)], cidx)
    pltpu.sync_copy(gid_hbm.at[pl.ds(s * 2, 2)], gidx)
    plsc.subcore_barrier()
    pltpu.sync_copy(h_buf.at[pl.ds(0, 128)], table.at[cidx.at[0]], add=True)
    pltpu.sync_copy(h_buf.at[pl.ds(128, 128)], table.at[cidx.at[1]], add=True)
    plsc.subcore_barrier()
    pltpu.async_copy(table.at[gidx.at[0]], vals_buf.at[pl.ds(0, 128)], sem).wait()
    pltpu.async_copy(table.at[gidx.at[1]], vals_buf.at[pl.ds(128, 128)], sem).wait()

    @pl.when(c == 0)
    def _():
        pltpu.sync_copy(vals_buf, va_hbm.at[pl.ds(s * SC_SCAT, SC_SCAT)])

    @pl.when(c == 1)
    def _():
        pltpu.sync_copy(vals_buf, vb_hbm.at[pl.ds(s * SC_SCAT, SC_SCAT)])


@functools.cache
def _get_pool():
    return pl.kernel(
        _pool_body,
        out_type=[jax.ShapeDtypeStruct((N, HHALF), jnp.float32),
                  jax.ShapeDtypeStruct((N, HHALF), jnp.float32)],
        mesh=plsc.VectorSubcoreMesh(core_axis_name="c", subcore_axis_name="s",
                                    num_cores=NC, num_subcores=NS),
        scratch_types=[
            pltpu.VMEM((SC_SCAT, HHALF), jnp.float32),
            pltpu.VMEM((SC_SCAT, HHALF), jnp.float32),
            pltpu.VMEM((2, 128), jnp.int32),
            pltpu.VMEM((2, 128), jnp.int32),
            pltpu.VMEM_SHARED((R_TAB, HHALF), jnp.float32),
            pltpu.SemaphoreType.DMA,
        ],
        name="social_pool_sc",
    )


_BLK = 1024


def _step_body(inp_ref, ha_ref, hb_ref, c_ref, va_ref, vb_ref, winT, wsT, wihT,
               whhT, bg, bi, bs, bo, woT, h2a_ref, h2b_ref, c2_ref, out_ref):
    f32 = jnp.float32
    r = jnp.maximum(
        jnp.dot(inp_ref[...], winT[...], preferred_element_type=f32) + bi[0:1, :], 0.0)
    vals = jnp.concatenate([va_ref[...], vb_ref[...]], axis=1)
    hs = jnp.concatenate([vals] * 9, axis=1)
    e = jnp.maximum(
        jnp.dot(hs, wsT[...], preferred_element_type=f32) + bs[0:1, :], 0.0)
    x = jnp.concatenate([r, e], axis=1)
    h = jnp.concatenate([ha_ref[...], hb_ref[...]], axis=1)
    g = (jnp.dot(x, wihT[...], preferred_element_type=f32)
         + jnp.dot(h, whhT[...], preferred_element_type=f32)
         + bg[0:1, :])
    ii = jax.nn.sigmoid(g[:, 0:HIDDEN])
    ff = jax.nn.sigmoid(g[:, HIDDEN:2 * HIDDEN])
    gg = jnp.tanh(g[:, 2 * HIDDEN:3 * HIDDEN])
    oo = jax.nn.sigmoid(g[:, 3 * HIDDEN:4 * HIDDEN])
    c2 = ff * c_ref[...] + ii * gg
    h2 = oo * jnp.tanh(c2)
    h2a_ref[...] = h2[:, :HHALF]
    h2b_ref[...] = h2[:, HHALF:]
    c2_ref[...] = c2
    out_ref[...] = jnp.dot(h2, woT[...], preferred_element_type=f32) + bo[0:1, :]


def _make_tc_step():
    data = lambda w: pl.BlockSpec((_BLK, w), lambda i: (i, 0))
    full = lambda a, b: pl.BlockSpec((a, b), lambda i: (0, 0))
    return pl.pallas_call(
        _step_body,
        grid=(N // _BLK,),
        in_specs=[
            data(PAD), data(HHALF), data(HHALF), data(HIDDEN),
            data(HHALF), data(HHALF),
            full(PAD, HIDDEN), full(9 * HIDDEN, SOCIAL),
            full(MEDIATE + SOCIAL, 4 * HIDDEN), full(HIDDEN, 4 * HIDDEN),
            full(PAD, 4 * HIDDEN), full(PAD, HIDDEN), full(PAD, SOCIAL),
            full(PAD, PAD), full(HIDDEN, PAD),
        ],
        out_specs=[data(HHALF), data(HHALF), data(HIDDEN), data(PAD)],
        out_shape=[
            jax.ShapeDtypeStruct((N, HHALF), jnp.float32),
            jax.ShapeDtypeStruct((N, HHALF), jnp.float32),
            jax.ShapeDtypeStruct((N, HIDDEN), jnp.float32),
            jax.ShapeDtypeStruct((N, PAD), jnp.float32),
        ],
        name="social_lstm_step_tc",
    )


_tc_step = functools.cache(_make_tc_step)


def kernel(X, part_masks, all_h_t, all_c_t, Y, T_obs, T_pred, W_in, b_in,
           W_soc, b_soc, W_ih, W_hh, b_ih, b_hh, W_out, b_out):
    f32 = jnp.float32
    cid, gid = _precompute_indices(X, part_masks)

    winT = jnp.zeros((PAD, HIDDEN), f32).at[:INPUT_DIM].set(W_in.T)
    wsT = W_soc.T
    wihT = W_ih.T
    whhT = W_hh.T
    bg = jnp.broadcast_to(b_ih + b_hh, (PAD, 4 * HIDDEN))
    bi = jnp.broadcast_to(b_in, (PAD, HIDDEN))
    bs = jnp.broadcast_to(b_soc, (PAD, SOCIAL))
    bo = jnp.zeros((PAD, PAD), f32).at[:, :OUT_DIM].set(
        jnp.broadcast_to(b_out, (PAD, OUT_DIM)))
    woT = jnp.zeros((HIDDEN, PAD), f32).at[:, :OUT_DIM].set(W_out.T)

    Xp = jnp.zeros((NSTEP, N, PAD), f32).at[:, :, :INPUT_DIM].set(X[:NSTEP])

    ha, hb = all_h_t[:, :HHALF], all_h_t[:, HHALF:]
    c = all_c_t
    outs = []
    for t in range(NSTEP):
        inp = Xp[min(t, 9)] if t <= 9 else outs[t - 2]
        va, vb = _get_pool()(ha, hb, cid[t], gid[t])
        ha, hb, c, out = _tc_step()(inp, ha, hb, c, va, vb, winT, wsT, wihT,
                                    whhT, bg, bi, bs, bo, woT)
        outs.append(out)
    outs.append(jnp.zeros((N, PAD), f32))
    res = jnp.stack(outs, axis=0)[:, :, :OUT_DIM]
    return res * part_masks[:, :, None]


# speedup vs baseline: 1.0058x; 1.0058x over previous
import functools

import jax
import jax.numpy as jnp
from jax import lax
from jax.experimental import pallas as pl
from jax.experimental.pallas import tpu as pltpu
from jax.experimental.pallas import tpu_sc as plsc

T_ALL, N = 20, 4096
NSTEP = 19
INPUT_DIM, HIDDEN, MEDIATE, OUT_DIM, SOCIAL = 2, 128, 128, 2, 16
N_SIZE = 2
CELL = 0.3
GRID = 256
PAD = 8

NC, NS = 2, 16
NW = NC * NS
ROWS_PER_TILE = 257
R_TAB = NS * ROWS_PER_TILE
DUMMY = N
SC_SCAT = N // NS
SC_GATH = N // NW


LUT_ROWS = 513
G_EMPTY = GRID * GRID


def _cell_keys(X, part_masks):
    Xs = X[:NSTEP]
    m = part_masks[:NSTEP]
    margin = 2 * N_SIZE * CELL
    x_min = Xs[:, :, 0].min(axis=1, keepdims=True) - margin
    y_min = Xs[:, :, 1].min(axis=1, keepdims=True) - margin
    px = jnp.floor((Xs[:, :, 0] - x_min) / CELL).astype(jnp.int32) * m.astype(jnp.int32)
    py = jnp.floor((Xs[:, :, 1] - y_min) / CELL).astype(jnp.int32) * m.astype(jnp.int32)
    px = jnp.clip(px, 0, GRID - 1)
    py = jnp.clip(py, 0, GRID - 1)
    key = px * GRID + py
    g = key - GRID - 1
    geff = jnp.where((m != 0) & (g >= 0), g, jnp.int32(G_EMPTY))
    return key, geff


def _route_body(key_hbm, geff_hbm, zlut_hbm, cid_hbm, gid_hbm,
                lut, keys, gbuf, cidb, gidb):
    c = lax.axis_index("c")
    s = lax.axis_index("s")
    wid = c * NS + s

    @pl.when(wid < NSTEP)
    def _():
        pltpu.sync_copy(zlut_hbm, lut)
        pltpu.sync_copy(key_hbm.at[wid], keys)
        pltpu.sync_copy(geff_hbm.at[wid], gbuf)

        def _split(idx):
            return [lax.shift_right_logical(idx, 7), lax.bitwise_and(idx, 127)]

        def scat(j, carry):
            idx = keys[j >> 3, pl.ds((j & 7) * 16, 16)]
            ids = lax.iota(jnp.int32, 16) + (j * 16 + 1)
            plsc.store_scatter(lut, _split(idx), ids)
            return carry

        lax.fori_loop(0, N // 16, scat, 0)

        def gath(j, carry):
            r, cs = j >> 3, (j & 7) * 16
            idx = keys[r, pl.ds(cs, 16)]
            cidb[r, pl.ds(cs, 16)] = plsc.load_gather(lut, _split(idx)) - 1
            gidx = gbuf[r, pl.ds(cs, 16)]
            rep = plsc.load_gather(lut, _split(gidx))
            gidb[r, pl.ds(cs, 16)] = jnp.where(rep == 0, jnp.int32(DUMMY), rep - 1)
            return carry

        lax.fori_loop(0, N // 16, gath, 0)
        pltpu.sync_copy(cidb, cid_hbm.at[wid])
        pltpu.sync_copy(gidb, gid_hbm.at[wid])


@functools.cache
def _get_route():
    return pl.kernel(
        _route_body,
        out_type=[jax.ShapeDtypeStruct((NSTEP, N // 128, 128), jnp.int32),
                  jax.ShapeDtypeStruct((NSTEP, N // 128, 128), jnp.int32)],
        mesh=plsc.VectorSubcoreMesh(core_axis_name="c", subcore_axis_name="s",
                                    num_cores=NC, num_subcores=NS),
        scratch_types=[
            pltpu.VMEM((LUT_ROWS, 128), jnp.int32),
            pltpu.VMEM((N // 128, 128), jnp.int32),
            pltpu.VMEM((N // 128, 128), jnp.int32),
            pltpu.VMEM((N // 128, 128), jnp.int32),
            pltpu.VMEM((N // 128, 128), jnp.int32),
        ],
        compiler_params=pltpu.CompilerParams(needs_layout_passes=False),
        name="social_route_sc",
    )


def _precompute_indices(X, part_masks):
    key, geff = _cell_keys(X, part_masks)
    zlut = jnp.zeros((LUT_ROWS, 128), jnp.int32)
    cid, gid = _get_route()(key.reshape(NSTEP, N // 128, 128),
                            geff.reshape(NSTEP, N // 128, 128), zlut)
    return cid, gid


HHALF = HIDDEN // NC


def _pool_body(ha_hbm, hb_hbm, cid_hbm, gid_hbm, zeros_hbm, va_hbm, vb_hbm,
               h_buf, vals_buf, cidx, gidx, table, sem):
    c = lax.axis_index("c")
    s = lax.axis_index("s")
    pltpu.sync_copy(zeros_hbm, table.at[pl.ds(s * ROWS_PER_TILE, ROWS_PER_TILE)])
    @pl.when(c == 0)
    def _():
        pltpu.sync_copy(ha_hbm.at[pl.ds(s * SC_SCAT, SC_SCAT)], h_buf)

    @pl.when(c == 1)
    def _():
        pltpu.sync_copy(hb_hbm.at[pl.ds(s * SC_SCAT, SC_SCAT)], h_buf)

    pltpu.sync_copy(cid_hbm.at[pl.ds(s * 2, 2)], cidx)
    pltpu.sync_copy(gid_hbm.at[pl.ds(s * 2, 2)], gidx)
    plsc.subcore_barrier()
    pltpu.sync_copy(h_buf.at[pl.ds(0, 128)], table.at[cidx.at[0]], add=True)
    pltpu.sync_copy(h_buf.at[pl.ds(128, 128)], table.at[cidx.at[1]], add=True)
    plsc.subcore_barrier()
    pltpu.async_copy(table.at[gidx.at[0]], vals_buf.at[pl.ds(0, 128)], sem).wait()
    pltpu.async_copy(table.at[gidx.at[1]], vals_buf.at[pl.ds(128, 128)], sem).wait()

    @pl.when(c == 0)
    def _():
        pltpu.sync_copy(vals_buf, va_hbm.at[pl.ds(s * SC_SCAT, SC_SCAT)])

    @pl.when(c == 1)
    def _():
        pltpu.sync_copy(vals_buf, vb_hbm.at[pl.ds(s * SC_SCAT, SC_SCAT)])


@functools.cache
def _get_pool():
    return pl.kernel(
        _pool_body,
        out_type=[jax.ShapeDtypeStruct((N, HHALF), jnp.float32),
                  jax.ShapeDtypeStruct((N, HHALF), jnp.float32)],
        mesh=plsc.VectorSubcoreMesh(core_axis_name="c", subcore_axis_name="s",
                                    num_cores=NC, num_subcores=NS),
        scratch_types=[
            pltpu.VMEM((SC_SCAT, HHALF), jnp.float32),
            pltpu.VMEM((SC_SCAT, HHALF), jnp.float32),
            pltpu.VMEM((2, 128), jnp.int32),
            pltpu.VMEM((2, 128), jnp.int32),
            pltpu.VMEM_SHARED((R_TAB, HHALF), jnp.float32),
            pltpu.SemaphoreType.DMA,
        ],
        name="social_pool_sc",
    )


_BLK = 1024


def _step_body(inp_ref, ha_ref, hb_ref, c_ref, va_ref, vb_ref, winT, wsT, wihT,
               whhT, bg, bi, bs, bo, woT, h2a_ref, h2b_ref, c2_ref, out_ref):
    f32 = jnp.float32
    r = jnp.maximum(
        jnp.dot(inp_ref[...], winT[...], preferred_element_type=f32) + bi[0:1, :], 0.0)
    vals = jnp.concatenate([va_ref[...], vb_ref[...]], axis=1)
    hs = jnp.concatenate([vals] * 9, axis=1)
    e = jnp.maximum(
        jnp.dot(hs, wsT[...], preferred_element_type=f32) + bs[0:1, :], 0.0)
    x = jnp.concatenate([r, e], axis=1)
    h = jnp.concatenate([ha_ref[...], hb_ref[...]], axis=1)
    g = (jnp.dot(x, wihT[...], preferred_element_type=f32)
         + jnp.dot(h, whhT[...], preferred_element_type=f32)
         + bg[0:1, :])
    ii = jax.nn.sigmoid(g[:, 0:HIDDEN])
    ff = jax.nn.sigmoid(g[:, HIDDEN:2 * HIDDEN])
    gg = jnp.tanh(g[:, 2 * HIDDEN:3 * HIDDEN])
    oo = jax.nn.sigmoid(g[:, 3 * HIDDEN:4 * HIDDEN])
    c2 = ff * c_ref[...] + ii * gg
    h2 = oo * jnp.tanh(c2)
    h2a_ref[...] = h2[:, :HHALF]
    h2b_ref[...] = h2[:, HHALF:]
    c2_ref[...] = c2
    out_ref[...] = jnp.dot(h2, woT[...], preferred_element_type=f32) + bo[0:1, :]


def _make_tc_step():
    data = lambda w: pl.BlockSpec((_BLK, w), lambda i: (i, 0))
    full = lambda a, b: pl.BlockSpec((a, b), lambda i: (0, 0))
    return pl.pallas_call(
        _step_body,
        grid=(N // _BLK,),
        in_specs=[
            data(PAD), data(HHALF), data(HHALF), data(HIDDEN),
            data(HHALF), data(HHALF),
            full(PAD, HIDDEN), full(9 * HIDDEN, SOCIAL),
            full(MEDIATE + SOCIAL, 4 * HIDDEN), full(HIDDEN, 4 * HIDDEN),
            full(PAD, 4 * HIDDEN), full(PAD, HIDDEN), full(PAD, SOCIAL),
            full(PAD, PAD), full(HIDDEN, PAD),
        ],
        out_specs=[data(HHALF), data(HHALF), data(HIDDEN), data(PAD)],
        out_shape=[
            jax.ShapeDtypeStruct((N, HHALF), jnp.float32),
            jax.ShapeDtypeStruct((N, HHALF), jnp.float32),
            jax.ShapeDtypeStruct((N, HIDDEN), jnp.float32),
            jax.ShapeDtypeStruct((N, PAD), jnp.float32),
        ],
        name="social_lstm_step_tc",
    )


_tc_step = functools.cache(_make_tc_step)


def kernel(X, part_masks, all_h_t, all_c_t, Y, T_obs, T_pred, W_in, b_in,
           W_soc, b_soc, W_ih, W_hh, b_ih, b_hh, W_out, b_out):
    f32 = jnp.float32
    cid, gid = _precompute_indices(X, part_masks)
    zeros_tab = jnp.zeros((ROWS_PER_TILE, HHALF), f32)

    winT = jnp.zeros((PAD, HIDDEN), f32).at[:INPUT_DIM].set(W_in.T)
    wsT = W_soc.T
    wihT = W_ih.T
    whhT = W_hh.T
    bg = jnp.broadcast_to(b_ih + b_hh, (PAD, 4 * HIDDEN))
    bi = jnp.broadcast_to(b_in, (PAD, HIDDEN))
    bs = jnp.broadcast_to(b_soc, (PAD, SOCIAL))
    bo = jnp.zeros((PAD, PAD), f32).at[:, :OUT_DIM].set(
        jnp.broadcast_to(b_out, (PAD, OUT_DIM)))
    woT = jnp.zeros((HIDDEN, PAD), f32).at[:, :OUT_DIM].set(W_out.T)

    Xp = jnp.zeros((NSTEP, N, PAD), f32).at[:, :, :INPUT_DIM].set(X[:NSTEP])

    ha, hb = all_h_t[:, :HHALF], all_h_t[:, HHALF:]
    c = all_c_t
    outs = []
    for t in range(NSTEP):
        inp = Xp[min(t, 9)] if t <= 9 else outs[t - 2]
        va, vb = _get_pool()(ha, hb, cid[t], gid[t], zeros_tab)
        ha, hb, c, out = _tc_step()(inp, ha, hb, c, va, vb, winT, wsT, wihT,
                                    whhT, bg, bi, bs, bo, woT)
        outs.append(out)
    outs.append(jnp.zeros((N, PAD), f32))
    res = jnp.stack(outs, axis=0)[:, :, :OUT_DIM]
    return res * part_masks[:, :, None]


# speedup vs baseline: 1.1525x; 1.1458x over previous
import functools

import jax
import jax.numpy as jnp
from jax import lax
from jax.experimental import pallas as pl
from jax.experimental.pallas import tpu as pltpu
from jax.experimental.pallas import tpu_sc as plsc

T_ALL, N = 20, 4096
NSTEP = 19
INPUT_DIM, HIDDEN, MEDIATE, OUT_DIM, SOCIAL = 2, 128, 128, 2, 16
N_SIZE = 2
CELL = 0.3
GRID = 256
PAD = 8

NC, NS = 2, 16
NW = NC * NS
ROWS_PER_TILE = 257
R_TAB = NS * ROWS_PER_TILE
DUMMY = N
SC_SCAT = N // NS
SC_GATH = N // NW


LUT_ROWS = 513
G_EMPTY = GRID * GRID


def _cell_keys(X, part_masks):
    Xs = X[:NSTEP]
    m = part_masks[:NSTEP]
    margin = 2 * N_SIZE * CELL
    x_min = Xs[:, :, 0].min(axis=1, keepdims=True) - margin
    y_min = Xs[:, :, 1].min(axis=1, keepdims=True) - margin
    px = jnp.floor((Xs[:, :, 0] - x_min) / CELL).astype(jnp.int32) * m.astype(jnp.int32)
    py = jnp.floor((Xs[:, :, 1] - y_min) / CELL).astype(jnp.int32) * m.astype(jnp.int32)
    px = jnp.clip(px, 0, GRID - 1)
    py = jnp.clip(py, 0, GRID - 1)
    key = px * GRID + py
    g = key - GRID - 1
    geff = jnp.where((m != 0) & (g >= 0), g, jnp.int32(G_EMPTY))
    return key, geff


def _route_body(key_hbm, geff_hbm, zlut_hbm, cid_hbm, gid_hbm,
                lut, keys, gbuf, cidb, gidb):
    c = lax.axis_index("c")
    s = lax.axis_index("s")
    wid = c * NS + s

    @pl.when(wid < NSTEP)
    def _():
        pltpu.sync_copy(zlut_hbm, lut)
        pltpu.sync_copy(key_hbm.at[wid], keys)
        pltpu.sync_copy(geff_hbm.at[wid], gbuf)

        def _split(idx):
            return [lax.shift_right_logical(idx, 7), lax.bitwise_and(idx, 127)]

        def scat(j, carry):
            idx = keys[j >> 3, pl.ds((j & 7) * 16, 16)]
            ids = lax.iota(jnp.int32, 16) + (j * 16 + 1)
            plsc.store_scatter(lut, _split(idx), ids)
            return carry

        lax.fori_loop(0, N // 16, scat, 0)

        def gath(j, carry):
            r, cs = j >> 3, (j & 7) * 16
            idx = keys[r, pl.ds(cs, 16)]
            cidb[r, pl.ds(cs, 16)] = plsc.load_gather(lut, _split(idx)) - 1
            gidx = gbuf[r, pl.ds(cs, 16)]
            rep = plsc.load_gather(lut, _split(gidx))
            gidb[r, pl.ds(cs, 16)] = jnp.where(rep == 0, jnp.int32(DUMMY), rep - 1)
            return carry

        lax.fori_loop(0, N // 16, gath, 0)
        pltpu.sync_copy(cidb, cid_hbm.at[wid])
        pltpu.sync_copy(gidb, gid_hbm.at[wid])


@functools.cache
def _get_route():
    return pl.kernel(
        _route_body,
        out_type=[jax.ShapeDtypeStruct((NSTEP, N // 128, 128), jnp.int32),
                  jax.ShapeDtypeStruct((NSTEP, N // 128, 128), jnp.int32)],
        mesh=plsc.VectorSubcoreMesh(core_axis_name="c", subcore_axis_name="s",
                                    num_cores=NC, num_subcores=NS),
        scratch_types=[
            pltpu.VMEM((LUT_ROWS, 128), jnp.int32),
            pltpu.VMEM((N // 128, 128), jnp.int32),
            pltpu.VMEM((N // 128, 128), jnp.int32),
            pltpu.VMEM((N // 128, 128), jnp.int32),
            pltpu.VMEM((N // 128, 128), jnp.int32),
        ],
        compiler_params=pltpu.CompilerParams(needs_layout_passes=False),
        name="social_route_sc",
    )


def _precompute_indices(X, part_masks):
    key, geff = _cell_keys(X, part_masks)
    zlut = jnp.zeros((LUT_ROWS, 128), jnp.int32)
    cid, gid = _get_route()(key.reshape(NSTEP, N // 128, 128),
                            geff.reshape(NSTEP, N // 128, 128), zlut)
    return cid, gid


def _pool_body(h_hbm, cid_hbm, gid_hbm, zeros_hbm, vals_hbm,
               h_buf, vals_buf, cidx, gidx, table, semz, sems):
    c = lax.axis_index("c")
    s = lax.axis_index("s")
    wid = c * NS + s
    z = pltpu.async_copy(
        zeros_hbm, table.at[pl.ds(s * ROWS_PER_TILE, ROWS_PER_TILE)], semz)
    a = pltpu.async_copy(h_hbm.at[pl.ds(s * SC_SCAT, SC_SCAT)], h_buf, sems)
    b = pltpu.async_copy(cid_hbm.at[pl.ds(s * 2, 2)], cidx, sems)
    d = pltpu.async_copy(gid_hbm.at[pl.ds(wid, 1)], gidx, sems)
    z.wait()
    a.wait()
    b.wait()
    d.wait()
    plsc.subcore_barrier()
    s1 = pltpu.async_copy(h_buf.at[pl.ds(0, 128)], table.at[cidx.at[0]],
                          sems, add=True)
    s2 = pltpu.async_copy(h_buf.at[pl.ds(128, 128)], table.at[cidx.at[1]],
                          sems, add=True)
    s1.wait()
    s2.wait()
    plsc.subcore_barrier()
    pltpu.async_copy(table.at[gidx.at[0]], vals_buf, sems).wait()
    pltpu.sync_copy(vals_buf, vals_hbm.at[pl.ds(wid * SC_GATH, SC_GATH)])


@functools.cache
def _get_pool():
    return pl.kernel(
        _pool_body,
        out_type=jax.ShapeDtypeStruct((N, HIDDEN), jnp.float32),
        mesh=plsc.VectorSubcoreMesh(core_axis_name="c", subcore_axis_name="s",
                                    num_cores=NC, num_subcores=NS),
        scratch_types=[
            pltpu.VMEM((SC_SCAT, HIDDEN), jnp.float32),
            pltpu.VMEM((SC_GATH, HIDDEN), jnp.float32),
            pltpu.VMEM((2, 128), jnp.int32),
            pltpu.VMEM((1, 128), jnp.int32),
            pltpu.VMEM_SHARED((R_TAB, HIDDEN), jnp.float32),
            pltpu.SemaphoreType.DMA,
            pltpu.SemaphoreType.DMA,
        ],
        name="social_pool_sc",
    )


_BLK = 1024


def _step_body(inp_ref, h_ref, c_ref, vals_ref, winT, wsT, wihT, whhT,
               bg, bi, bs, bo, woT, h2_ref, c2_ref, out_ref):
    f32 = jnp.float32
    r = jnp.maximum(
        jnp.dot(inp_ref[...], winT[...], preferred_element_type=f32) + bi[0:1, :], 0.0)
    vals = vals_ref[...]
    hs = jnp.concatenate([vals] * 9, axis=1)
    e = jnp.maximum(
        jnp.dot(hs, wsT[...], preferred_element_type=f32) + bs[0:1, :], 0.0)
    x = jnp.concatenate([r, e], axis=1)
    g = (jnp.dot(x, wihT[...], preferred_element_type=f32)
         + jnp.dot(h_ref[...], whhT[...], preferred_element_type=f32)
         + bg[0:1, :])
    ii = jax.nn.sigmoid(g[:, 0:HIDDEN])
    ff = jax.nn.sigmoid(g[:, HIDDEN:2 * HIDDEN])
    gg = jnp.tanh(g[:, 2 * HIDDEN:3 * HIDDEN])
    oo = jax.nn.sigmoid(g[:, 3 * HIDDEN:4 * HIDDEN])
    c2 = ff * c_ref[...] + ii * gg
    h2 = oo * jnp.tanh(c2)
    h2_ref[...] = h2
    c2_ref[...] = c2
    out_ref[...] = jnp.dot(h2, woT[...], preferred_element_type=f32) + bo[0:1, :]


def _make_tc_step():
    data = lambda w: pl.BlockSpec((_BLK, w), lambda i: (i, 0))
    full = lambda a, b: pl.BlockSpec((a, b), lambda i: (0, 0))
    return pl.pallas_call(
        _step_body,
        grid=(N // _BLK,),
        in_specs=[
            data(PAD), data(HIDDEN), data(HIDDEN), data(HIDDEN),
            full(PAD, HIDDEN), full(9 * HIDDEN, SOCIAL),
            full(MEDIATE + SOCIAL, 4 * HIDDEN), full(HIDDEN, 4 * HIDDEN),
            full(PAD, 4 * HIDDEN), full(PAD, HIDDEN), full(PAD, SOCIAL),
            full(PAD, PAD), full(HIDDEN, PAD),
        ],
        out_specs=[data(HIDDEN), data(HIDDEN), data(PAD)],
        out_shape=[
            jax.ShapeDtypeStruct((N, HIDDEN), jnp.float32),
            jax.ShapeDtypeStruct((N, HIDDEN), jnp.float32),
            jax.ShapeDtypeStruct((N, PAD), jnp.float32),
        ],
        name="social_lstm_step_tc",
    )


_tc_step = functools.cache(_make_tc_step)


def kernel(X, part_masks, all_h_t, all_c_t, Y, T_obs, T_pred, W_in, b_in,
           W_soc, b_soc, W_ih, W_hh, b_ih, b_hh, W_out, b_out):
    f32 = jnp.float32
    cid, gid = _precompute_indices(X, part_masks)
    zeros_tab = jnp.zeros((ROWS_PER_TILE, HIDDEN), f32)

    winT = jnp.zeros((PAD, HIDDEN), f32).at[:INPUT_DIM].set(W_in.T)
    wsT = W_soc.T
    wihT = W_ih.T
    whhT = W_hh.T
    bg = jnp.broadcast_to(b_ih + b_hh, (PAD, 4 * HIDDEN))
    bi = jnp.broadcast_to(b_in, (PAD, HIDDEN))
    bs = jnp.broadcast_to(b_soc, (PAD, SOCIAL))
    bo = jnp.zeros((PAD, PAD), f32).at[:, :OUT_DIM].set(
        jnp.broadcast_to(b_out, (PAD, OUT_DIM)))
    woT = jnp.zeros((HIDDEN, PAD), f32).at[:, :OUT_DIM].set(W_out.T)

    Xp = jnp.zeros((NSTEP, N, PAD), f32).at[:, :, :INPUT_DIM].set(X[:NSTEP])

    h, c = all_h_t, all_c_t
    outs = []
    for t in range(NSTEP):
        inp = Xp[min(t, 9)] if t <= 9 else outs[t - 2]
        vals = _get_pool()(h, cid[t], gid[t], zeros_tab)
        h, c, out = _tc_step()(inp, h, c, vals, winT, wsT, wihT, whhT,
                               bg, bi, bs, bo, woT)
        outs.append(out)
    outs.append(jnp.zeros((N, PAD), f32))
    res = jnp.stack(outs, axis=0)[:, :, :OUT_DIM]
    return res * part_masks[:, :, None]
